# baseline (speedup 1.0000x reference)
import functools

import jax
import jax.numpy as jnp
from jax import lax
from jax.experimental import pallas as pl
from jax.experimental.pallas import tpu as pltpu
from jax.experimental.pallas import tpu_sc as plsc

VOCAB = 1000000
HALF_V = VOCAB // 2
EMBED = 64
BATCH = 16384
POS = 10
NEG = 50

NC = 2
NS = 16
NW = NC * NS
L = 16

B_PER_W = BATCH // NW
CB = 8
STEPS = B_PER_W // CB
NEG_CHUNKS = CB * NEG // 80

O_CEN = 0
O_POS = CB
O_NEG = CB + CB * POS
O_OFF = 512
PACK_W = 1024
NCHUNK = BATCH // CB

T_BLK = 8192
N_TBLK = 62
H_PAIR = N_TBLK * T_BLK


def _tc_transpose_kernel(in_a, out_a, in_b, out_b, eye_ref, comb_ref):
    dn = (((0,), (0,)), ((), ()))
    eye = eye_ref[...]

    def pack32(ref):
        y = lax.dot_general(ref[...], eye, dn,
                            preferred_element_type=jnp.float32)
        b16 = lax.bitcast_convert_type(y.astype(jnp.bfloat16), jnp.uint16)
        lo = b16[:, : EMBED // 2].astype(jnp.int32)
        hi = b16[:, EMBED // 2 :].astype(jnp.int32)
        return lo | (hi << 16)

    comb_ref[...] = jnp.concatenate(
        [pack32(in_a), pack32(out_a), pack32(in_b), pack32(out_b)], axis=1)


def _sc_dots_kernel(pack_hbm, comb_hbm, dots_hbm,
                    idx_v0, idx_v1, cen0, cen1, pos0, pos1, neg0, neg1,
                    dots0, dots1, sem0, sem1):
    wid = lax.axis_index("s") * NC + lax.axis_index("c")
    lane = lax.broadcasted_iota(jnp.int32, (L,), 0)
    bufs = [(idx_v0, cen0, pos0, neg0, dots0, sem0),
            (idx_v1, cen1, pos1, neg1, dots1, sem1)]

    def stage(chunk, buf):
        idx_v, cen_rows, pos_rows, neg_rows, _, sem = bufs[buf]
        pltpu.sync_copy(pack_hbm.at[pl.ds(chunk * PACK_W, PACK_W)], idx_v)
        pltpu.async_copy(comb_hbm.at[idx_v.at[pl.ds(O_CEN, CB)]],
                         cen_rows, sem)
        pltpu.async_copy(comb_hbm.at[idx_v.at[pl.ds(O_POS, CB * POS)]],
                         pos_rows, sem)
        for k in range(NEG_CHUNKS):
            pltpu.async_copy(
                comb_hbm.at[idx_v.at[pl.ds(O_NEG + 80 * k, 80)]],
                neg_rows.at[pl.ds(80 * k, 80)], sem)

    def drain(buf):
        idx_v, cen_rows, pos_rows, neg_rows, _, sem = bufs[buf]
        pltpu.make_async_copy(comb_hbm.at[idx_v.at[pl.ds(O_CEN, CB)]],
                              cen_rows, sem).wait()
        pltpu.make_async_copy(comb_hbm.at[idx_v.at[pl.ds(O_POS, CB * POS)]],
                              pos_rows, sem).wait()
        for k in range(NEG_CHUNKS):
            pltpu.make_async_copy(
                comb_hbm.at[idx_v.at[pl.ds(O_NEG + 80 * k, 80)]],
                neg_rows.at[pl.ds(80 * k, 80)], sem).wait()

    def compute(s, buf):
        idx_v, cen_rows, pos_rows, neg_rows, dots_v, _ = bufs[buf]
        b0 = (wid * STEPS + s) * CB

        def unpack2(ref, row, base):
            u = ref[row, pl.ds(base, L)]
            return plsc.unpack(plsc.bitcast(u, jnp.bfloat16),
                               format=plsc.PackFormat.INTERLEAVED)

        def item(b, carry):
            cboff = idx_v[pl.ds(O_OFF + O_CEN + b, L)][0]
            c = (unpack2(cen_rows, b, cboff)
                 + unpack2(cen_rows, b, cboff + L))
            poffs = idx_v[pl.ds(O_OFF + O_POS + b * POS, L)]
            noffs = [idx_v[pl.ds(O_OFF + O_NEG + b * NEG + L * t, L)]
                     for t in range(4)]
            d = [jnp.zeros((L,), jnp.float32) for _ in range(4)]
            for j in range(POS):
                row = b * POS + j
                bo = poffs[j] + 2 * L
                ra, rb = unpack2(pos_rows, row, bo)
                rc, rd = unpack2(pos_rows, row, bo + L)
                acc = ra * c[0] + rb * c[1] + rc * c[2] + rd * c[3]
                dot = jnp.sum(acc)
                g, ln = divmod(j, L)
                d[g] = jnp.where(lane == ln, dot, d[g])
            for j in range(NEG):
                row = b * NEG + j
                bo = noffs[j // L][j % L] + 2 * L
                ra, rb = unpack2(neg_rows, row, bo)
                rc, rd = unpack2(neg_rows, row, bo + L)
                acc = ra * c[0] + rb * c[1] + rc * c[2] + rd * c[3]
                dot = jnp.sum(acc)
                g, ln = divmod(POS + j, L)
                d[g] = jnp.where(lane == ln, dot, d[g])
            for g in range(4):
                dots_v[pl.ds(b * EMBED + L * g, L)] = d[g]
            return carry

        lax.fori_loop(0, CB, item, 0)
        pltpu.sync_copy(dots_v, dots_hbm.at[pl.ds(b0 * EMBED, CB * EMBED)])

    stage(wid * STEPS, 0)

    def two_steps(s2, carry):
        s = s2 * 2
        stage(wid * STEPS + s + 1, 1)
        drain(0)
        compute(s, 0)

        @pl.when(s + 2 < STEPS)
        def _():
            stage(wid * STEPS + s + 2, 0)

        drain(1)
        compute(s + 1, 1)
        return carry

    lax.fori_loop(0, STEPS // 2, two_steps, 0)


def _tc_loss_kernel(dots_ref, out_ref):
    x = dots_ref[...]
    lane = lax.broadcasted_iota(jnp.int32, x.shape, 1)
    m = lax.rem(lane, EMBED)
    sign = jnp.where(m < POS, 1.0, -1.0).astype(jnp.float32)
    y = jax.nn.log_sigmoid(x * sign)
    y = jnp.where(m < POS + NEG, y, 0.0)
    s0 = -jnp.sum(y[:, :EMBED], axis=1, keepdims=True)
    s1 = -jnp.sum(y[:, EMBED:], axis=1, keepdims=True)
    out_ref[...] = jnp.concatenate([s0, s1], axis=1)


def kernel(cen_tensor, pos_tensors, neg_tensors, in_table, out_table):
    cen = cen_tensor.reshape(NCHUNK, CB)
    pos = pos_tensors.reshape(NCHUNK, CB * POS)
    neg = neg_tensors.reshape(NCHUNK, CB * NEG)

    def halves(x):
        return jnp.where(x < H_PAIR, x, x - H_PAIR), \
               jnp.where(x < H_PAIR, 0, EMBED)

    (cen_g, cen_o), (pos_g, pos_o), (neg_g, neg_o) = map(halves,
                                                         (cen, pos, neg))
    pad = jnp.zeros((NCHUNK, PACK_W // 2 - O_NEG - CB * NEG), jnp.int32)
    packed = jnp.concatenate(
        [cen_g, pos_g, neg_g, pad, cen_o, pos_o, neg_o, pad],
        axis=1).reshape(-1)

    in_t = in_table.T
    out_t = out_table.T
    def bmap(i):
        return (0, jnp.minimum(i, N_TBLK - 2) + N_TBLK)

    comb = pl.pallas_call(
        _tc_transpose_kernel,
        grid=(N_TBLK,),
        in_specs=[pl.BlockSpec((EMBED, T_BLK), lambda i: (0, i)),
                  pl.BlockSpec((EMBED, T_BLK), lambda i: (0, i)),
                  pl.BlockSpec((EMBED, T_BLK), bmap),
                  pl.BlockSpec((EMBED, T_BLK), bmap),
                  pl.BlockSpec((EMBED, EMBED), lambda i: (0, 0))],
        out_specs=pl.BlockSpec((T_BLK, 2 * EMBED), lambda i: (i, 0)),
        out_shape=jax.ShapeDtypeStruct((H_PAIR, 2 * EMBED), jnp.int32),
    )(in_t, out_t, in_t, out_t, jnp.eye(EMBED, dtype=jnp.float32))

    mesh = plsc.VectorSubcoreMesh(core_axis_name="c", subcore_axis_name="s")
    sc_call = functools.partial(
        pl.kernel, mesh=mesh,
        compiler_params=pltpu.CompilerParams(needs_layout_passes=False),
        out_type=jax.ShapeDtypeStruct((BATCH * EMBED,), jnp.float32),
        scratch_types=[
            pltpu.VMEM((PACK_W,), jnp.int32),
            pltpu.VMEM((PACK_W,), jnp.int32),
            pltpu.VMEM((CB, 2 * EMBED), jnp.int32),
            pltpu.VMEM((CB, 2 * EMBED), jnp.int32),
            pltpu.VMEM((CB * POS, 2 * EMBED), jnp.int32),
            pltpu.VMEM((CB * POS, 2 * EMBED), jnp.int32),
            pltpu.VMEM((CB * NEG, 2 * EMBED), jnp.int32),
            pltpu.VMEM((CB * NEG, 2 * EMBED), jnp.int32),
            pltpu.VMEM((CB * EMBED,), jnp.float32),
            pltpu.VMEM((CB * EMBED,), jnp.float32),
            pltpu.SemaphoreType.DMA,
            pltpu.SemaphoreType.DMA,
        ],
    )(_sc_dots_kernel)
    dots = sc_call(packed, comb)

    loss2 = pl.pallas_call(
        _tc_loss_kernel,
        out_shape=jax.ShapeDtypeStruct((BATCH // 2, 2), jnp.float32),
    )(dots.reshape(BATCH // 2, 2 * EMBED))
    return loss2.reshape(BATCH)


# speedup vs baseline: 1.3470x; 1.3470x over previous
import functools

import jax
import jax.numpy as jnp
from jax import lax
from jax.experimental import pallas as pl
from jax.experimental.pallas import tpu as pltpu
from jax.experimental.pallas import tpu_sc as plsc

VOCAB = 1000000
EMBED = 64
BATCH = 16384
POS = 10
NEG = 50

NC = 2
NS = 16
NW = NC * NS
L = 16

B_PER_W = BATCH // NW
CB = 8
STEPS = B_PER_W // CB
NEG_CHUNKS = CB * NEG // 80

O_CEN = 0
O_POS = CB
O_NEG = CB + CB * POS
PACK_W = 512
NCHUNK = BATCH // CB

T_BLK = 16384


def _tc_transpose_kernel(in_ref, out_ref, eye_ref, comb_ref):
    dn = (((0,), (0,)), ((), ()))
    eye = eye_ref[...]
    comb_ref[:, :EMBED] = lax.dot_general(
        in_ref[...], eye, dn, preferred_element_type=jnp.float32)
    comb_ref[:, EMBED:] = lax.dot_general(
        out_ref[...], eye, dn, preferred_element_type=jnp.float32)


def _sc_dots_kernel(pack_hbm, comb_hbm, dots_hbm,
                    idx_v0, idx_v1, cen0, cen1, pos0, pos1, neg0, neg1,
                    dots0, dots1, sem0, sem1):
    wid = lax.axis_index("s") * NC + lax.axis_index("c")
    lane = lax.broadcasted_iota(jnp.int32, (L,), 0)
    bufs = [(idx_v0, cen0, pos0, neg0, dots0, sem0),
            (idx_v1, cen1, pos1, neg1, dots1, sem1)]

    def stage(chunk, buf):
        idx_v, cen_rows, pos_rows, neg_rows, _, sem = bufs[buf]
        pltpu.sync_copy(pack_hbm.at[pl.ds(chunk * PACK_W, PACK_W)], idx_v)
        pltpu.async_copy(comb_hbm.at[idx_v.at[pl.ds(O_CEN, CB)]],
                         cen_rows, sem)
        pltpu.async_copy(comb_hbm.at[idx_v.at[pl.ds(O_POS, CB * POS)]],
                         pos_rows, sem)
        for k in range(NEG_CHUNKS):
            pltpu.async_copy(
                comb_hbm.at[idx_v.at[pl.ds(O_NEG + 80 * k, 80)]],
                neg_rows.at[pl.ds(80 * k, 80)], sem)

    def drain(buf):
        idx_v, cen_rows, pos_rows, neg_rows, _, sem = bufs[buf]
        pltpu.make_async_copy(comb_hbm.at[idx_v.at[pl.ds(O_CEN, CB)]],
                              cen_rows, sem).wait()
        pltpu.make_async_copy(comb_hbm.at[idx_v.at[pl.ds(O_POS, CB * POS)]],
                              pos_rows, sem).wait()
        for k in range(NEG_CHUNKS):
            pltpu.make_async_copy(
                comb_hbm.at[idx_v.at[pl.ds(O_NEG + 80 * k, 80)]],
                neg_rows.at[pl.ds(80 * k, 80)], sem).wait()

    def compute(s, buf):
        _, cen_rows, pos_rows, neg_rows, dots_v, _ = bufs[buf]
        b0 = (wid * STEPS + s) * CB

        def item(b, carry):
            c = [cen_rows[b, pl.ds(L * k, L)] for k in range(4)]
            d = [jnp.zeros((L,), jnp.float32) for _ in range(4)]
            for j in range(POS):
                row = b * POS + j
                acc = pos_rows[row, pl.ds(EMBED, L)] * c[0]
                for k in range(1, 4):
                    acc = acc + pos_rows[row, pl.ds(EMBED + L * k, L)] * c[k]
                dot = jnp.sum(acc)
                g, ln = divmod(j, L)
                d[g] = jnp.where(lane == ln, dot, d[g])
            for j in range(NEG):
                row = b * NEG + j
                acc = neg_rows[row, pl.ds(EMBED, L)] * c[0]
                for k in range(1, 4):
                    acc = acc + neg_rows[row, pl.ds(EMBED + L * k, L)] * c[k]
                dot = jnp.sum(acc)
                g, ln = divmod(POS + j, L)
                d[g] = jnp.where(lane == ln, dot, d[g])
            for g in range(4):
                dots_v[pl.ds(b * EMBED + L * g, L)] = d[g]
            return carry

        lax.fori_loop(0, CB, item, 0)
        pltpu.sync_copy(dots_v, dots_hbm.at[pl.ds(b0 * EMBED, CB * EMBED)])

    stage(wid * STEPS, 0)

    def two_steps(s2, carry):
        s = s2 * 2
        stage(wid * STEPS + s + 1, 1)
        drain(0)
        compute(s, 0)

        @pl.when(s + 2 < STEPS)
        def _():
            stage(wid * STEPS + s + 2, 0)

        drain(1)
        compute(s + 1, 1)
        return carry

    lax.fori_loop(0, STEPS // 2, two_steps, 0)


def _tc_loss_kernel(dots_ref, out_ref):
    x = dots_ref[...]
    lane = lax.broadcasted_iota(jnp.int32, x.shape, 1)
    m = lax.rem(lane, EMBED)
    sign = jnp.where(m < POS, 1.0, -1.0).astype(jnp.float32)
    y = jax.nn.log_sigmoid(x * sign)
    y = jnp.where(m < POS + NEG, y, 0.0)
    s0 = -jnp.sum(y[:, :EMBED], axis=1, keepdims=True)
    s1 = -jnp.sum(y[:, EMBED:], axis=1, keepdims=True)
    out_ref[...] = jnp.concatenate([s0, s1], axis=1)


def kernel(cen_tensor, pos_tensors, neg_tensors, in_table, out_table):
    cen = cen_tensor.reshape(NCHUNK, CB)
    pos = pos_tensors.reshape(NCHUNK, CB * POS)
    neg = neg_tensors.reshape(NCHUNK, CB * NEG)
    packed = jnp.concatenate(
        [cen, pos, neg,
         jnp.zeros((NCHUNK, PACK_W - O_NEG - CB * NEG), jnp.int32)],
        axis=1).reshape(-1)

    in_t = in_table.T
    out_t = out_table.T
    n_tblk = (VOCAB + T_BLK - 1) // T_BLK
    comb = pl.pallas_call(
        _tc_transpose_kernel,
        grid=(n_tblk,),
        in_specs=[pl.BlockSpec((EMBED, T_BLK), lambda i: (0, i)),
                  pl.BlockSpec((EMBED, T_BLK), lambda i: (0, i)),
                  pl.BlockSpec((EMBED, EMBED), lambda i: (0, 0))],
        out_specs=pl.BlockSpec((T_BLK, 2 * EMBED), lambda i: (i, 0)),
        out_shape=jax.ShapeDtypeStruct((VOCAB, 2 * EMBED), jnp.float32),
    )(in_t, out_t, jnp.eye(EMBED, dtype=jnp.float32))

    mesh = plsc.VectorSubcoreMesh(core_axis_name="c", subcore_axis_name="s")
    sc_call = functools.partial(
        pl.kernel, mesh=mesh,
        compiler_params=pltpu.CompilerParams(needs_layout_passes=False),
        out_type=jax.ShapeDtypeStruct((BATCH * EMBED,), jnp.float32),
        scratch_types=[
            pltpu.VMEM((PACK_W,), jnp.int32),
            pltpu.VMEM((PACK_W,), jnp.int32),
            pltpu.VMEM((CB, 2 * EMBED), jnp.float32),
            pltpu.VMEM((CB, 2 * EMBED), jnp.float32),
            pltpu.VMEM((CB * POS, 2 * EMBED), jnp.float32),
            pltpu.VMEM((CB * POS, 2 * EMBED), jnp.float32),
            pltpu.VMEM((CB * NEG, 2 * EMBED), jnp.float32),
            pltpu.VMEM((CB * NEG, 2 * EMBED), jnp.float32),
            pltpu.VMEM((CB * EMBED,), jnp.float32),
            pltpu.VMEM((CB * EMBED,), jnp.float32),
            pltpu.SemaphoreType.DMA,
            pltpu.SemaphoreType.DMA,
        ],
    )(_sc_dots_kernel)
    dots = sc_call(packed, comb)

    loss2 = pl.pallas_call(
        _tc_loss_kernel,
        out_shape=jax.ShapeDtypeStruct((BATCH // 2, 2), jnp.float32),
    )(dots.reshape(BATCH // 2, 2 * EMBED))
    return loss2.reshape(BATCH)
